# baseline (speedup 1.0000x reference)
import jax
import jax.numpy as jnp
from jax.experimental import pallas as pl
from jax.experimental.pallas import tpu as pltpu

B, N, D = 512, 12, 128
E = N * N
BB = 32
NB = B // BB

_DNT = (((1,), (1,)), ((), ()))


def _graph_module_kernel(gfull_ref, fwv_ref, fbv_ref, awv_ref, abv_ref,
                         epw_ref, epb_ref, bng_ref, bnb_ref,
                         out_ref, norm_ref, g2_ref, b2_ref):
    i = pl.program_id(0)

    @pl.when(i == 0)
    def _prologue():
        gf = gfull_ref[:, 0, :]
        gv = jax.lax.dot_general(
            gf, fwv_ref[:], _DNT,
            preferred_element_type=jnp.float32) + fbv_ref[:]
        h = jax.lax.dot_general(
            gv, awv_ref[:], _DNT,
            preferred_element_type=jnp.float32) + abv_ref[:]
        ev = jax.lax.dot_general(
            h, epw_ref[:], _DNT,
            preferred_element_type=jnp.float32) + epb_ref[:]
        mean = jnp.mean(ev)
        var = jnp.mean((ev - mean) ** 2)
        norm_ref[:] = (ev - mean) * jax.lax.rsqrt(var + 1e-5)
        g2_ref[:] = jnp.broadcast_to(bng_ref[0][:, None], (E, D))
        b2_ref[:] = jnp.broadcast_to(bnb_ref[0][:, None], (E, D))

    blk = norm_ref[pl.ds(i * BB, BB), :]
    out_ref[:] = jnp.maximum(
        blk[:, None, :] * g2_ref[:][None, :, :] + b2_ref[:][None, :, :],
        0.0)


@jax.jit
def kernel(node_feature, global_feature, fam_wq, fam_bq, fam_wk, fam_bk,
           fam_wv, fam_bv, arm_wq, arm_bq, arm_wk, arm_bk, arm_wv, arm_bv,
           ep_w, ep_b, bn_g, bn_b):
    fbv = fam_bv.reshape(1, D)
    abv = arm_bv.reshape(1, D)
    epb = ep_b.reshape(1, D)
    bng = bn_g.reshape(1, E)
    bnb = bn_b.reshape(1, E)

    gspec = pl.BlockSpec((B, 1, D), lambda i: (0, 0, 0))
    wspec = pl.BlockSpec((D, D), lambda i: (0, 0))
    bspec = pl.BlockSpec((1, D), lambda i: (0, 0))
    bnspec = pl.BlockSpec((1, E), lambda i: (0, 0))

    out = pl.pallas_call(
        _graph_module_kernel,
        grid=(NB,),
        in_specs=[gspec, wspec, bspec, wspec, bspec, wspec, bspec,
                  bnspec, bnspec],
        out_specs=pl.BlockSpec((BB, E, D), lambda i: (i, 0, 0)),
        out_shape=jax.ShapeDtypeStruct((B, E, D), jnp.float32),
        scratch_shapes=[pltpu.VMEM((B, D), jnp.float32),
                        pltpu.VMEM((E, D), jnp.float32),
                        pltpu.VMEM((E, D), jnp.float32)],
        compiler_params=pltpu.CompilerParams(
            dimension_semantics=("arbitrary",)),
    )(global_feature, fam_wv, fbv, arm_wv, abv, ep_w, epb, bng, bnb)
    return out


# speedup vs baseline: 1.0302x; 1.0302x over previous
import jax
import jax.numpy as jnp
from jax.experimental import pallas as pl
from jax.experimental.pallas import tpu as pltpu

B, N, D = 512, 12, 128
E = N * N
BB = 128
NB = B // BB

_DNT = (((1,), (1,)), ((), ()))


def _graph_module_kernel(gfull_ref, fwv_ref, fbv_ref, awv_ref, abv_ref,
                         epw_ref, epb_ref, bng_ref, bnb_ref,
                         out_ref, norm_ref, g2_ref, b2_ref):
    i = pl.program_id(0)

    @pl.when(i == 0)
    def _prologue():
        gf = gfull_ref[:, 0, :]
        gv = jax.lax.dot_general(
            gf, fwv_ref[:], _DNT,
            preferred_element_type=jnp.float32) + fbv_ref[:]
        h = jax.lax.dot_general(
            gv, awv_ref[:], _DNT,
            preferred_element_type=jnp.float32) + abv_ref[:]
        ev = jax.lax.dot_general(
            h, epw_ref[:], _DNT,
            preferred_element_type=jnp.float32) + epb_ref[:]
        mean = jnp.mean(ev)
        var = jnp.mean((ev - mean) ** 2)
        norm_ref[:] = (ev - mean) * jax.lax.rsqrt(var + 1e-5)
        g2_ref[:] = jnp.broadcast_to(bng_ref[0][:, None], (E, D))
        b2_ref[:] = jnp.broadcast_to(bnb_ref[0][:, None], (E, D))

    blk = norm_ref[pl.ds(i * BB, BB), :]
    out_ref[:] = jnp.maximum(
        blk[:, None, :] * g2_ref[:][None, :, :] + b2_ref[:][None, :, :],
        0.0)


@jax.jit
def kernel(node_feature, global_feature, fam_wq, fam_bq, fam_wk, fam_bk,
           fam_wv, fam_bv, arm_wq, arm_bq, arm_wk, arm_bk, arm_wv, arm_bv,
           ep_w, ep_b, bn_g, bn_b):
    fbv = fam_bv.reshape(1, D)
    abv = arm_bv.reshape(1, D)
    epb = ep_b.reshape(1, D)
    bng = bn_g.reshape(1, E)
    bnb = bn_b.reshape(1, E)

    gspec = pl.BlockSpec((B, 1, D), lambda i: (0, 0, 0))
    wspec = pl.BlockSpec((D, D), lambda i: (0, 0))
    bspec = pl.BlockSpec((1, D), lambda i: (0, 0))
    bnspec = pl.BlockSpec((1, E), lambda i: (0, 0))

    out = pl.pallas_call(
        _graph_module_kernel,
        grid=(NB,),
        in_specs=[gspec, wspec, bspec, wspec, bspec, wspec, bspec,
                  bnspec, bnspec],
        out_specs=pl.BlockSpec((BB, E, D), lambda i: (i, 0, 0)),
        out_shape=jax.ShapeDtypeStruct((B, E, D), jnp.float32),
        scratch_shapes=[pltpu.VMEM((B, D), jnp.float32),
                        pltpu.VMEM((E, D), jnp.float32),
                        pltpu.VMEM((E, D), jnp.float32)],
        compiler_params=pltpu.CompilerParams(
            dimension_semantics=("arbitrary",)),
    )(global_feature, fam_wv, fbv, arm_wv, abv, ep_w, epb, bng, bnb)
    return out


# speedup vs baseline: 1.1258x; 1.0928x over previous
import jax
import jax.numpy as jnp
from jax.experimental import pallas as pl
from jax.experimental.pallas import tpu as pltpu

B, N, D = 512, 12, 128
E = N * N
BB = 64
NB = B // BB

_DNT = (((1,), (1,)), ((), ()))


def _graph_module_kernel(gfull_ref, fwv_ref, fbv_ref, awv_ref, abv_ref,
                         epw_ref, epb_ref, bng_ref, bnb_ref,
                         out_ref, norm_ref, g2_ref, b2_ref):
    i = pl.program_id(0)

    @pl.when(i == 0)
    def _prologue():
        pa = jnp.dot(epw_ref[:], awv_ref[:],
                     preferred_element_type=jnp.float32)
        w = jnp.dot(pa, fwv_ref[:], preferred_element_type=jnp.float32)
        fa = jax.lax.dot_general(
            fbv_ref[:], awv_ref[:], _DNT,
            preferred_element_type=jnp.float32) + abv_ref[:]
        c = jax.lax.dot_general(
            fa, epw_ref[:], _DNT,
            preferred_element_type=jnp.float32) + epb_ref[:]
        gf = gfull_ref[:, 0, :]
        ev = jax.lax.dot_general(
            gf, w, _DNT, preferred_element_type=jnp.float32) + c
        mean = jnp.mean(ev)
        var = jnp.mean((ev - mean) ** 2)
        norm_ref[:] = (ev - mean) * jax.lax.rsqrt(var + 1e-5)
        g2_ref[:] = jnp.broadcast_to(bng_ref[0][:, None], (E, D))
        b2_ref[:] = jnp.broadcast_to(bnb_ref[0][:, None], (E, D))

    blk = norm_ref[pl.ds(i * BB, BB), :]
    out_ref[:] = jnp.maximum(
        blk[:, None, :] * g2_ref[:][None, :, :] + b2_ref[:][None, :, :],
        0.0)


@jax.jit
def kernel(node_feature, global_feature, fam_wq, fam_bq, fam_wk, fam_bk,
           fam_wv, fam_bv, arm_wq, arm_bq, arm_wk, arm_bk, arm_wv, arm_bv,
           ep_w, ep_b, bn_g, bn_b):
    fbv = fam_bv.reshape(1, D)
    abv = arm_bv.reshape(1, D)
    epb = ep_b.reshape(1, D)
    bng = bn_g.reshape(1, E)
    bnb = bn_b.reshape(1, E)

    gspec = pl.BlockSpec((B, 1, D), lambda i: (0, 0, 0))
    wspec = pl.BlockSpec((D, D), lambda i: (0, 0))
    bspec = pl.BlockSpec((1, D), lambda i: (0, 0))
    bnspec = pl.BlockSpec((1, E), lambda i: (0, 0))

    out = pl.pallas_call(
        _graph_module_kernel,
        grid=(NB,),
        in_specs=[gspec, wspec, bspec, wspec, bspec, wspec, bspec,
                  bnspec, bnspec],
        out_specs=pl.BlockSpec((BB, E, D), lambda i: (i, 0, 0)),
        out_shape=jax.ShapeDtypeStruct((B, E, D), jnp.float32),
        scratch_shapes=[pltpu.VMEM((B, D), jnp.float32),
                        pltpu.VMEM((E, D), jnp.float32),
                        pltpu.VMEM((E, D), jnp.float32)],
        compiler_params=pltpu.CompilerParams(
            dimension_semantics=("arbitrary",)),
    )(global_feature, fam_wv, fbv, arm_wv, abv, ep_w, epb, bng, bnb)
    return out
